# baseline (speedup 1.0000x reference)
import functools

import jax
import jax.numpy as jnp
from jax import lax
from jax.experimental import pallas as pl
from jax.experimental.pallas import tpu as pltpu
from jax.experimental.pallas import tpu_sc as plsc

OLD_V = 1000000
NEW_V = 1024
D = 64
B = 4096
S = 200
SP = 256
NC = 2
NS = 16
NW = NC * NS
NXR = B // NW
NBUF = 4
G1 = 128
G2 = S - G1
GOFF = tuple(range(0, S - 16 + 1, 16)) + (S - 16,)


def _body(x_hbm, old_hbm, new_hbm, out_hbm, xb, idxb, rows, newt_v,
          xsem, gsem, osem):
    wid = lax.axis_index("s") * NC + lax.axis_index("c")
    row0 = pl.multiple_of(wid * NXR, NXR)
    pltpu.sync_copy(new_hbm, newt_v)
    lane = lax.iota(jnp.int32, 16)

    def xdma_desc(r, b):
        return pltpu.make_async_copy(
            x_hbm.at[row0 + r, pl.ds(0, S)],
            xb.at[pl.ds(pl.multiple_of(b * S, S), S)], xsem.at[b])

    def g1_desc(r, b):
        bs = pl.multiple_of(b * S, S)
        return pltpu.make_async_copy(
            old_hbm.at[idxb.at[pl.ds(bs, G1)]],
            rows.at[b, pl.ds(0, G1)], gsem.at[b])

    def g2_desc(r, b):
        bs = pl.multiple_of(b * S, S)
        return pltpu.make_async_copy(
            old_hbm.at[idxb.at[pl.ds(bs + G1, G2)]],
            rows.at[b, pl.ds(G1, G2)], gsem.at[b])

    def store_desc(r, b):
        return pltpu.make_async_copy(rows.at[b], out_hbm.at[row0 + r],
                                     osem.at[b])

    def fire_x(r):
        xdma_desc(r, lax.rem(r, NBUF)).start()

    def fire_gather(r):
        b = lax.rem(r, NBUF)
        bs = pl.multiple_of(b * S, S)
        for go in GOFF:
            xv = xb[pl.ds(bs + go, 16)]
            idxb[pl.ds(bs + go, 16)] = jnp.minimum(xv, OLD_V - 1)
        g1_desc(r, b).start()
        g2_desc(r, b).start()

    fire_x(jnp.int32(0))
    fire_x(jnp.int32(1))
    fire_x(jnp.int32(2))
    for r0 in range(2):
        r = jnp.int32(r0)
        xdma_desc(r, lax.rem(r, NBUF)).wait()
        fire_gather(r)

    def step(r, carry0):
        @pl.when(r + 3 < NXR)
        def _():
            fire_x(r + 3)

        @pl.when(r + 2 < NXR)
        def _():
            r2 = r + 2
            b2 = lax.rem(r2, NBUF)
            xdma_desc(r2, b2).wait()

            @pl.when(r >= 2)
            def _():
                store_desc(r - 2, b2).wait()
            fire_gather(r2)

        b = lax.rem(r, NBUF)
        bs = pl.multiple_of(b * S, S)
        g1_desc(r, b).wait()
        g2_desc(r, b).wait()
        bsp = jnp.full((16,), b, jnp.int32)
        for go in GOFF:
            xv = xb[pl.ds(bs + go, 16)]
            m = xv >= OLD_V
            cnt = plsc.all_reduce_population_count(m)

            @pl.when(cnt[0] > 0)
            def _():
                nid = jnp.maximum(xv - OLD_V, 0)
                tok = lane + go
                for d in range(D):
                    col = jnp.full((16,), d, jnp.int32)
                    vals = plsc.load_gather(newt_v, [nid, col])
                    plsc.store_scatter(rows, [bsp, tok, col], vals, mask=m)
        store_desc(r, b).start()
        return carry0

    lax.fori_loop(0, NXR, step, 0)

    for k in range(NBUF):
        r = jnp.int32(NXR - NBUF + k)
        store_desc(r, lax.rem(r, NBUF)).wait()


def kernel(x, old_table, new_table):
    xp = jnp.pad(x, ((0, 0), (0, SP - S)))
    mesh = plsc.VectorSubcoreMesh(core_axis_name="c", subcore_axis_name="s")
    run = functools.partial(
        pl.kernel,
        mesh=mesh,
        out_type=jax.ShapeDtypeStruct((B, S, D), jnp.float32),
        scratch_types=[
            pltpu.VMEM((NBUF * S,), jnp.int32),
            pltpu.VMEM((NBUF * S,), jnp.int32),
            pltpu.VMEM((NBUF, S, D), jnp.float32),
            pltpu.VMEM((NEW_V, D), jnp.float32),
            pltpu.SemaphoreType.DMA((NBUF,)),
            pltpu.SemaphoreType.DMA((NBUF,)),
            pltpu.SemaphoreType.DMA((NBUF,)),
        ],
        compiler_params=pltpu.CompilerParams(
            needs_layout_passes=False, use_tc_tiling_on_sc=False),
    )(_body)
    return run(xp, old_table, new_table)


# speedup vs baseline: 1.3150x; 1.3150x over previous
import functools

import jax
import jax.numpy as jnp
from jax import lax
from jax.experimental import pallas as pl
from jax.experimental.pallas import tpu as pltpu
from jax.experimental.pallas import tpu_sc as plsc

OLD_V = 1000000
NEW_V = 1024
D = 64
B = 4096
S = 200
SP = 256
NC = 2
NS = 16
NW = NC * NS
NXR = B // NW
NBUF = 4
G1 = 128
G2 = S - G1
GOFF = tuple(range(0, S - 16 + 1, 16)) + (S - 16,)


def _body(x_hbm, old_hbm, new_hbm, out_hbm, xb, idxb, rows, newt_v,
          xsem, gsem, osem):
    wid = lax.axis_index("s") * NC + lax.axis_index("c")
    row0 = pl.multiple_of(wid * NXR, NXR)
    pltpu.sync_copy(new_hbm, newt_v)
    lane = lax.iota(jnp.int32, 16)

    def xdma_desc(r, b):
        return pltpu.make_async_copy(
            x_hbm.at[row0 + r, pl.ds(0, S)],
            xb.at[pl.ds(pl.multiple_of(b * S, S), S)], xsem.at[b])

    def g1_desc(r, b):
        bs = pl.multiple_of(b * S, S)
        return pltpu.make_async_copy(
            old_hbm.at[idxb.at[pl.ds(bs, G1)]],
            rows.at[b, pl.ds(0, G1)], gsem.at[b])

    def g2_desc(r, b):
        bs = pl.multiple_of(b * S, S)
        return pltpu.make_async_copy(
            old_hbm.at[idxb.at[pl.ds(bs + G1, G2)]],
            rows.at[b, pl.ds(G1, G2)], gsem.at[b])

    def store_desc(r, b):
        return pltpu.make_async_copy(
            rows.at[b], out_hbm.at[row0 + r, pl.ds(0, S), pl.ds(0, D)],
            osem.at[b])

    def fire_x(r):
        xdma_desc(r, lax.rem(r, NBUF)).start()

    def fire_gather(r):
        b = lax.rem(r, NBUF)
        bs = pl.multiple_of(b * S, S)
        for go in GOFF:
            xv = xb[pl.ds(bs + go, 16)]
            idxb[pl.ds(bs + go, 16)] = jnp.minimum(xv, OLD_V - 1)
        g1_desc(r, b).start()
        g2_desc(r, b).start()

    fire_x(jnp.int32(0))
    fire_x(jnp.int32(1))
    fire_x(jnp.int32(2))
    for r0 in range(2):
        r = jnp.int32(r0)
        xdma_desc(r, lax.rem(r, NBUF)).wait()
        fire_gather(r)

    def step(r, carry0):
        @pl.when(r + 3 < NXR)
        def _():
            fire_x(r + 3)

        @pl.when(r + 2 < NXR)
        def _():
            r2 = r + 2
            b2 = lax.rem(r2, NBUF)
            xdma_desc(r2, b2).wait()

            @pl.when(r >= 2)
            def _():
                store_desc(r - 2, b2).wait()
            fire_gather(r2)

        b = lax.rem(r, NBUF)
        bs = pl.multiple_of(b * S, S)
        g1_desc(r, b).wait()
        g2_desc(r, b).wait()
        bsp = jnp.full((16,), b, jnp.int32)
        for go in GOFF:
            xv = xb[pl.ds(bs + go, 16)]
            m = xv >= OLD_V
            cnt = plsc.all_reduce_population_count(m)

            @pl.when(cnt[0] > 0)
            def _():
                nid = jnp.maximum(xv - OLD_V, 0)
                tok = lane + go
                for d in range(D):
                    col = jnp.full((16,), d, jnp.int32)
                    vals = plsc.load_gather(newt_v, [nid, col])
                    plsc.store_scatter(rows, [bsp, tok, col], vals, mask=m)
        store_desc(r, b).start()
        return carry0

    lax.fori_loop(0, NXR, step, 0)

    for k in range(NBUF):
        r = jnp.int32(NXR - NBUF + k)
        store_desc(r, lax.rem(r, NBUF)).wait()


def kernel(x, old_table, new_table):
    xp = jnp.pad(x, ((0, 0), (0, SP - S)))
    mesh = plsc.VectorSubcoreMesh(core_axis_name="c", subcore_axis_name="s")
    run = functools.partial(
        pl.kernel,
        mesh=mesh,
        out_type=jax.ShapeDtypeStruct((B, S, 2 * D), jnp.float32),
        scratch_types=[
            pltpu.VMEM((NBUF * S,), jnp.int32),
            pltpu.VMEM((NBUF * S,), jnp.int32),
            pltpu.VMEM((NBUF, S, D), jnp.float32),
            pltpu.VMEM((NEW_V, D), jnp.float32),
            pltpu.SemaphoreType.DMA((NBUF,)),
            pltpu.SemaphoreType.DMA((NBUF,)),
            pltpu.SemaphoreType.DMA((NBUF,)),
        ],
        compiler_params=pltpu.CompilerParams(
            needs_layout_passes=False, use_tc_tiling_on_sc=False),
    )(_body)
    return run(xp, old_table, new_table)[..., :D]
